# baseline (speedup 1.0000x reference)
import functools

import jax
import jax.numpy as jnp
from jax import lax
from jax.experimental import pallas as pl
from jax.experimental.pallas import tpu as pltpu
from jax.experimental.pallas import tpu_sc as plsc

_VOCAB = 1000000
_EMBED = 32
_NUM_CLASSES = 50
_PAD_IDX = 0
_BATCH = 4096
_SEQ = 200

_NC = 2
_NS = 16
_NW = _NC * _NS
_B_PER_W = _BATCH // _NW
_C0, _C1 = 120, 80
_LANES = 16
_RING = 4

_DT_BLK = 32768
_DT_GRID = (_VOCAB + _DT_BLK - 1) // _DT_BLK
_DT_Q = _DT_BLK // 8
_TBL_ROWS = _DT_GRID * _DT_BLK
_BLK_MASK = jnp.int32(-_DT_BLK)
_Q_MASK = jnp.int32(_DT_Q - 1)
_Q_SHIFT = _DT_Q.bit_length() - 1


def _tc_detile(tT_ref, out_ref):
    t = tT_ref[...]
    u = lax.bitcast_convert_type(t.astype(jnp.bfloat16), jnp.uint16)
    u32 = u.astype(jnp.uint32).reshape(_EMBED // 2, 2, _DT_BLK)
    packed_u = u32[:, 0, :] | (u32[:, 1, :] << 16)
    packed = lax.bitcast_convert_type(packed_u, jnp.float32)
    tbig = jnp.concatenate(
        [packed[:, p * _DT_Q:(p + 1) * _DT_Q] for p in range(8)], axis=0)
    out_ref[...] = tbig.T


def _detile(tableT):
    return pl.pallas_call(
        _tc_detile,
        grid=(_DT_GRID,),
        in_specs=[pl.BlockSpec((_EMBED, _DT_BLK), lambda i: (0, i))],
        out_specs=pl.BlockSpec((_DT_Q, 128), lambda i: (i, 0)),
        out_shape=jax.ShapeDtypeStruct((_DT_GRID * _DT_Q, 128),
                                       jnp.float32),
    )(tableT)


_mesh = plsc.VectorSubcoreMesh(core_axis_name="c", subcore_axis_name="s")


@functools.partial(
    pl.kernel,
    mesh=_mesh,
    out_type=jax.ShapeDtypeStruct((_BATCH, _EMBED), jnp.float32),
    compiler_params=pltpu.CompilerParams(use_tc_tiling_on_sc=False,
                                         needs_layout_passes=False),
    name="sc_gather_sum",
    scratch_types=[
        pltpu.VMEM((_B_PER_W, _SEQ), jnp.int32),
        pltpu.VMEM((_SEQ, 16), jnp.float32),
        pltpu.VMEM((_SEQ, 16), jnp.float32),
        pltpu.VMEM((_SEQ, 16), jnp.float32),
        pltpu.VMEM((_SEQ, 16), jnp.float32),
        pltpu.VMEM((_B_PER_W, _EMBED), jnp.float32),
        pltpu.SemaphoreType.DMA,
        pltpu.SemaphoreType.DMA,
        pltpu.SemaphoreType.DMA,
        pltpu.SemaphoreType.DMA,
    ],
)
def _sc_sum(x_hbm, table_hbm, out_hbm, idx_v, buf0, buf1, buf2, buf3,
            sums_v, sem0, sem1, sem2, sem3):
    wid = lax.axis_index("s") * _NC + lax.axis_index("c")
    base = wid * _B_PER_W

    pltpu.sync_copy(x_hbm.at[pl.ds(base, _B_PER_W), :], idx_v)

    _cols = tuple(range(0, _SEQ - 15, 16)) + (_SEQ - 16,)
    _tail_lo = _SEQ - (_SEQ % 16)

    def to_slot(r, carry):
        for c in _cols:
            v = idx_v[r, pl.ds(c, 16)]
            g = ((v & _BLK_MASK)
                 | ((v & _Q_MASK) << 3)
                 | ((v >> _Q_SHIFT) & jnp.int32(7)))
            if c + 16 > _tail_lo and c != _tail_lo - 16:
                keep = lax.iota(jnp.int32, 16) < jnp.int32(_tail_lo - c)
                g = jnp.where(keep, v, g)
            idx_v[r, pl.ds(c, 16)] = g
        return carry

    lax.fori_loop(0, _B_PER_W, to_slot, 0)

    bufs = (buf0, buf1, buf2, buf3)
    sems = (sem0, sem1, sem2, sem3)

    def issue(r, buf, sem):
        pltpu.async_copy(table_hbm.at[idx_v.at[r, pl.ds(0, _C0)]],
                         buf.at[pl.ds(0, _C0)], sem)
        pltpu.async_copy(table_hbm.at[idx_v.at[r, pl.ds(_C0, _C1)]],
                         buf.at[pl.ds(_C0, _C1)], sem)

    def wait(r, buf, sem):
        pltpu.make_async_copy(table_hbm.at[idx_v.at[r, pl.ds(0, _C0)]],
                              buf.at[pl.ds(0, _C0)], sem).wait()
        pltpu.make_async_copy(table_hbm.at[idx_v.at[r, pl.ds(_C0, _C1)]],
                              buf.at[pl.ds(_C0, _C1)], sem).wait()

    for r0 in range(_RING - 1):
        issue(r0, bufs[r0], sems[r0])

    def outer(i, carry):
        for p in range(_RING):
            r = i * _RING + p
            nxt = r + _RING - 1

            @pl.when(nxt < _B_PER_W)
            def _():
                issue(nxt, bufs[(p + _RING - 1) % _RING],
                      sems[(p + _RING - 1) % _RING])

            wait(r, bufs[p], sems[p])
            buf = bufs[p]

            def accum(j, acc):
                a0, a1 = acc
                for u in range(8):
                    row = j * 8 + u
                    pk = plsc.bitcast(buf[row, pl.ds(0, _LANES)],
                                      jnp.bfloat16)
                    ea, eb = plsc.unpack(pk, format=plsc.PackFormat.INTERLEAVED)
                    a0 = a0 + ea
                    a1 = a1 + eb
                return (a0, a1)

            zero = jnp.zeros((_LANES,), jnp.float32)
            a0, a1 = lax.fori_loop(0, _SEQ // 8, accum, (zero, zero))
            sums_v[r, pl.ds(0, _LANES)] = a0
            sums_v[r, pl.ds(_LANES, _LANES)] = a1
        return carry

    lax.fori_loop(0, _B_PER_W // _RING, outer, 0)
    pltpu.sync_copy(sums_v, out_hbm.at[pl.ds(base, _B_PER_W)])


def _tc_head(summed_ref, x_ref, w_ref, b_ref, out_ref):
    xb = x_ref[...]
    cnt = jnp.sum((xb != _PAD_IDX).astype(jnp.float32), axis=1, keepdims=True)
    cnt = jnp.maximum(cnt, 1.0)
    avg = summed_ref[...] / cnt
    out_ref[...] = (
        jnp.dot(avg, w_ref[...], preferred_element_type=jnp.float32)
        + b_ref[...]
    )


_TC_BLK = 512

import numpy as _np
_W_PERM = _np.concatenate([_np.arange(0, 32, 2), _np.arange(1, 32, 2)])


def kernel(x, table, W, b):
    table_rm = _detile(table.T).reshape(_TBL_ROWS, 16)
    summed = _sc_sum(x, table_rm)
    w_perm = W[_W_PERM, :]
    grid = (_BATCH // _TC_BLK,)
    out = pl.pallas_call(
        _tc_head,
        grid=grid,
        in_specs=[
            pl.BlockSpec((_TC_BLK, _EMBED), lambda i: (i, 0)),
            pl.BlockSpec((_TC_BLK, _SEQ), lambda i: (i, 0)),
            pl.BlockSpec((_EMBED, _NUM_CLASSES), lambda i: (0, 0)),
            pl.BlockSpec((1, _NUM_CLASSES), lambda i: (0, 0)),
        ],
        out_specs=pl.BlockSpec((_TC_BLK, _NUM_CLASSES), lambda i: (i, 0)),
        out_shape=jax.ShapeDtypeStruct((_BATCH, _NUM_CLASSES), jnp.float32),
    )(summed, x, w_perm, b.reshape(1, _NUM_CLASSES))
    return out


# speedup vs baseline: 1.1021x; 1.1021x over previous
import functools

import jax
import jax.numpy as jnp
from jax import lax
from jax.experimental import pallas as pl
from jax.experimental.pallas import tpu as pltpu
from jax.experimental.pallas import tpu_sc as plsc

_VOCAB = 1000000
_EMBED = 32
_NUM_CLASSES = 50
_PAD_IDX = 0
_BATCH = 4096
_SEQ = 200

_NC = 2
_NS = 16
_NW = _NC * _NS
_B_PER_W = _BATCH // _NW
_C0, _C1 = 120, 80
_LANES = 16
_RING = 8

_DT_BLK = 65536
_DT_GRID = (_VOCAB + _DT_BLK - 1) // _DT_BLK
_DT_Q = _DT_BLK // 4
_TBL_ROWS = _DT_GRID * _DT_BLK
_BLK_MASK = jnp.int32(-_DT_BLK)
_Q_MASK = jnp.int32(_DT_Q - 1)
_Q_SHIFT = _DT_Q.bit_length() - 1


def _tc_detile(tT_ref, out_ref):
    t = tT_ref[...]
    tbig = jnp.concatenate(
        [t[:, p * _DT_Q:(p + 1) * _DT_Q] for p in range(4)], axis=0)
    out_ref[...] = tbig.T


def _detile(tableT):
    return pl.pallas_call(
        _tc_detile,
        grid=(_DT_GRID,),
        in_specs=[pl.BlockSpec((_EMBED, _DT_BLK), lambda i: (0, i))],
        out_specs=pl.BlockSpec((_DT_Q, 4 * _EMBED), lambda i: (i, 0)),
        out_shape=jax.ShapeDtypeStruct((_DT_GRID * _DT_Q, 4 * _EMBED),
                                       jnp.float32),
    )(tableT)


_mesh = plsc.VectorSubcoreMesh(core_axis_name="c", subcore_axis_name="s")


@functools.partial(
    pl.kernel,
    mesh=_mesh,
    out_type=jax.ShapeDtypeStruct((_BATCH, _EMBED), jnp.float32),
    compiler_params=pltpu.CompilerParams(use_tc_tiling_on_sc=False),
    name="sc_gather_sum",
    scratch_types=[
        pltpu.VMEM((_B_PER_W, _SEQ), jnp.int32),
        pltpu.VMEM((_SEQ, _EMBED), jnp.float32),
        pltpu.VMEM((_SEQ, _EMBED), jnp.float32),
        pltpu.VMEM((_SEQ, _EMBED), jnp.float32),
        pltpu.VMEM((_SEQ, _EMBED), jnp.float32),
        pltpu.VMEM((_SEQ, _EMBED), jnp.float32),
        pltpu.VMEM((_SEQ, _EMBED), jnp.float32),
        pltpu.VMEM((_SEQ, _EMBED), jnp.float32),
        pltpu.VMEM((_SEQ, _EMBED), jnp.float32),
        pltpu.VMEM((_B_PER_W, _EMBED), jnp.float32),
        pltpu.SemaphoreType.DMA,
        pltpu.SemaphoreType.DMA,
        pltpu.SemaphoreType.DMA,
        pltpu.SemaphoreType.DMA,
        pltpu.SemaphoreType.DMA,
        pltpu.SemaphoreType.DMA,
        pltpu.SemaphoreType.DMA,
        pltpu.SemaphoreType.DMA,
    ],
)
def _sc_sum(x_hbm, table_hbm, out_hbm, idx_v, buf0, buf1, buf2, buf3,
            buf4, buf5, buf6, buf7, sums_v,
            sem0, sem1, sem2, sem3, sem4, sem5, sem6, sem7):
    wid = lax.axis_index("s") * _NC + lax.axis_index("c")
    base = wid * _B_PER_W

    pltpu.sync_copy(x_hbm.at[pl.ds(base, _B_PER_W), :], idx_v)

    _cols = tuple(range(0, _SEQ - 15, 16)) + (_SEQ - 16,)
    _tail_lo = _SEQ - (_SEQ % 16)

    def to_slot(r, carry):
        for c in _cols:
            v = idx_v[r, pl.ds(c, 16)]
            g = ((v & _BLK_MASK)
                 | ((v & _Q_MASK) << 2)
                 | ((v >> _Q_SHIFT) & jnp.int32(3)))
            if c + 16 > _tail_lo and c != _tail_lo - 16:
                keep = lax.iota(jnp.int32, 16) < jnp.int32(_tail_lo - c)
                g = jnp.where(keep, v, g)
            idx_v[r, pl.ds(c, 16)] = g
        return carry

    lax.fori_loop(0, _B_PER_W, to_slot, 0)

    bufs = (buf0, buf1, buf2, buf3, buf4, buf5, buf6, buf7)
    sems = (sem0, sem1, sem2, sem3, sem4, sem5, sem6, sem7)

    def issue(r, buf, sem):
        pltpu.async_copy(table_hbm.at[idx_v.at[r, pl.ds(0, _C0)]],
                         buf.at[pl.ds(0, _C0)], sem)
        pltpu.async_copy(table_hbm.at[idx_v.at[r, pl.ds(_C0, _C1)]],
                         buf.at[pl.ds(_C0, _C1)], sem)

    def wait(r, buf, sem):
        pltpu.make_async_copy(table_hbm.at[idx_v.at[r, pl.ds(0, _C0)]],
                              buf.at[pl.ds(0, _C0)], sem).wait()
        pltpu.make_async_copy(table_hbm.at[idx_v.at[r, pl.ds(_C0, _C1)]],
                              buf.at[pl.ds(_C0, _C1)], sem).wait()

    for r0 in range(_RING - 1):
        issue(r0, bufs[r0], sems[r0])

    def outer(i, carry):
        for p in range(_RING):
            r = i * _RING + p
            nxt = r + _RING - 1

            @pl.when(nxt < _B_PER_W)
            def _():
                issue(nxt, bufs[(p + _RING - 1) % _RING],
                      sems[(p + _RING - 1) % _RING])

            wait(r, bufs[p], sems[p])
            buf = bufs[p]

            def accum(j, acc):
                a0, a1 = acc
                for u in range(8):
                    row = j * 8 + u
                    a0 = a0 + buf[row, pl.ds(0, _LANES)]
                    a1 = a1 + buf[row, pl.ds(_LANES, _LANES)]
                return (a0, a1)

            zero = jnp.zeros((_LANES,), jnp.float32)
            a0, a1 = lax.fori_loop(0, _SEQ // 8, accum, (zero, zero))
            sums_v[r, pl.ds(0, _LANES)] = a0
            sums_v[r, pl.ds(_LANES, _LANES)] = a1
        return carry

    lax.fori_loop(0, _B_PER_W // _RING, outer, 0)
    pltpu.sync_copy(sums_v, out_hbm.at[pl.ds(base, _B_PER_W)])


def _tc_head(summed_ref, x_ref, w_ref, b_ref, out_ref):
    xb = x_ref[...]
    cnt = jnp.sum((xb != _PAD_IDX).astype(jnp.float32), axis=1, keepdims=True)
    cnt = jnp.maximum(cnt, 1.0)
    avg = summed_ref[...] / cnt
    out_ref[...] = (
        jnp.dot(avg, w_ref[...], preferred_element_type=jnp.float32)
        + b_ref[...]
    )


_TC_BLK = 512


def kernel(x, table, W, b):
    table_rm = _detile(table.T).reshape(_TBL_ROWS, _EMBED)
    summed = _sc_sum(x, table_rm)
    grid = (_BATCH // _TC_BLK,)
    out = pl.pallas_call(
        _tc_head,
        grid=grid,
        in_specs=[
            pl.BlockSpec((_TC_BLK, _EMBED), lambda i: (i, 0)),
            pl.BlockSpec((_TC_BLK, _SEQ), lambda i: (i, 0)),
            pl.BlockSpec((_EMBED, _NUM_CLASSES), lambda i: (0, 0)),
            pl.BlockSpec((1, _NUM_CLASSES), lambda i: (0, 0)),
        ],
        out_specs=pl.BlockSpec((_TC_BLK, _NUM_CLASSES), lambda i: (i, 0)),
        out_shape=jax.ShapeDtypeStruct((_BATCH, _NUM_CLASSES), jnp.float32),
    )(summed, x, W, b.reshape(1, _NUM_CLASSES))
    return out
